# Initial kernel scaffold; baseline (speedup 1.0000x reference)
import jax
import jax.numpy as jnp
from jax.experimental import pallas as pl


def kernel(center_words, context_words, neg_samples, in_embed_w, out_embed_w):
    raise NotImplementedError("write your pallas kernel here")


# speedup vs baseline: 5.1235x; 5.1235x over previous
import functools

import jax
import jax.numpy as jnp
from jax import lax
from jax.experimental import pallas as pl
from jax.experimental.pallas import tpu as pltpu
from jax.experimental.pallas import tpu_sc as plsc

NC = 2
NS = 16
NW = NC * NS
L = 16

K1 = 21
CB = 32
IDX_SUB = 112


def _make_sc_call(B, V, D):
    assert D == 64
    assert B % (NW * CB) == 0
    b_per_w = B // NW
    n_chunks = b_per_w // CB
    rows_per_chunk = CB * K1
    n_sub = rows_per_chunk // IDX_SUB
    assert n_sub * IDX_SUB == rows_per_chunk

    mesh = plsc.VectorSubcoreMesh(
        core_axis_name="c", subcore_axis_name="s", num_cores=NC, num_subcores=NS
    )

    @functools.partial(
        pl.kernel,
        out_type=jax.ShapeDtypeStruct((B * K1,), jnp.float32),
        mesh=mesh,
        scratch_types=[
            pltpu.VMEM((CB,), jnp.int32),
            pltpu.VMEM((rows_per_chunk,), jnp.int32),
            pltpu.VMEM((CB, D), jnp.float32),
            pltpu.VMEM((rows_per_chunk, D), jnp.float32),
            pltpu.VMEM((rows_per_chunk,), jnp.float32),
            pltpu.SemaphoreType.DMA,
        ],
        compiler_params=pltpu.CompilerParams(use_tc_tiling_on_sc=False),
    )
    def sc_call(in_w_hbm, out_w_hbm, cidx_hbm, comb_hbm, out_hbm,
                cidx_v, idx_v, crows_v, rows_v, scores_v, sem):
        wid = lax.axis_index("s") * NC + lax.axis_index("c")
        lane = lax.iota(jnp.int32, L)
        perms = [(lane ^ (1 << e))[:, None] for e in range(4)]
        gdims = lax.GatherDimensionNumbers(
            offset_dims=(), collapsed_slice_dims=(0,), start_index_map=(0,)
        )

        def shuffle(p, pm):
            return lax.gather(
                p, pm, gdims, (1,), mode=lax.GatherScatterMode.PROMISE_IN_BOUNDS
            )

        def lane_sum(p):
            for pm in perms:
                p = p + shuffle(p, pm)
            return p

        def chunk_body(t, carry):
            base = wid * b_per_w + t * CB
            base21 = base * K1

            pltpu.sync_copy(cidx_hbm.at[pl.ds(base, CB)], cidx_v)
            pltpu.sync_copy(comb_hbm.at[pl.ds(base21, rows_per_chunk)], idx_v)

            copies = [pltpu.async_copy(in_w_hbm.at[cidx_v], crows_v, sem)]
            for j in range(n_sub):
                copies.append(
                    pltpu.async_copy(
                        out_w_hbm.at[idx_v.at[pl.ds(j * IDX_SUB, IDX_SUB)]],
                        rows_v.at[pl.ds(j * IDX_SUB, IDX_SUB)],
                        sem,
                    )
                )
            for c in copies:
                c.wait()

            def b_body(bi, carry2):
                c0 = crows_v[bi, pl.ds(0, L)]
                c1 = crows_v[bi, pl.ds(L, L)]
                c2 = crows_v[bi, pl.ds(2 * L, L)]
                c3 = crows_v[bi, pl.ds(3 * L, L)]
                q0 = bi * K1
                acc0 = jnp.zeros((L,), jnp.float32)
                acc1 = jnp.zeros((L,), jnp.float32)
                for r in range(K1):
                    q = q0 + r
                    p = (rows_v[q, pl.ds(0, L)] * c0
                         + rows_v[q, pl.ds(L, L)] * c1
                         + rows_v[q, pl.ds(2 * L, L)] * c2
                         + rows_v[q, pl.ds(3 * L, L)] * c3)
                    s = lane_sum(p)
                    if r < L:
                        acc0 = jnp.where(lane == r, s, acc0)
                    if r >= K1 - L:
                        acc1 = jnp.where(lane == (r - (K1 - L)), s, acc1)
                scores_v[pl.ds(q0, L)] = acc0
                scores_v[pl.ds(q0 + (K1 - L), L)] = acc1
                return carry2

            lax.fori_loop(0, CB, b_body, 0, unroll=False)

            pltpu.sync_copy(scores_v, out_hbm.at[pl.ds(base21, rows_per_chunk)])
            return carry

        lax.fori_loop(0, n_chunks, chunk_body, 0, unroll=False)

    return sc_call


def kernel(center_words, context_words, neg_samples, in_embed_w, out_embed_w):
    B = center_words.shape[0]
    V, D = in_embed_w.shape
    cidx = center_words.astype(jnp.int32)
    comb = jnp.concatenate(
        [context_words.astype(jnp.int32)[:, None], neg_samples.astype(jnp.int32)],
        axis=1,
    ).reshape(-1)
    sc_call = _make_sc_call(B, V, D)
    scores = sc_call(in_embed_w, out_embed_w, cidx, comb).reshape(B, K1)
    return scores[:, :1], scores[:, 1:]


# speedup vs baseline: 5.1444x; 1.0041x over previous
import functools

import jax
import jax.numpy as jnp
from jax import lax
from jax.experimental import pallas as pl
from jax.experimental.pallas import tpu as pltpu
from jax.experimental.pallas import tpu_sc as plsc

NC = 2
NS = 16
NW = NC * NS
L = 16

K = 20
CB = 32
NEG_SUB = 128


def _make_sc_call(B, V, D):
    assert D == 64
    assert B % (NW * CB) == 0
    b_per_w = B // NW
    n_chunks = b_per_w // CB
    negs_per_chunk = CB * K
    n_sub = negs_per_chunk // NEG_SUB
    assert n_sub * NEG_SUB == negs_per_chunk
    n_grp = CB // L

    mesh = plsc.VectorSubcoreMesh(
        core_axis_name="c", subcore_axis_name="s", num_cores=NC, num_subcores=NS
    )

    @functools.partial(
        pl.kernel,
        out_type=(
            jax.ShapeDtypeStruct((B,), jnp.float32),
            jax.ShapeDtypeStruct((B * K,), jnp.float32),
        ),
        mesh=mesh,
        scratch_types=[
            pltpu.VMEM((CB,), jnp.int32),
            pltpu.VMEM((CB,), jnp.int32),
            pltpu.VMEM((negs_per_chunk,), jnp.int32),
            pltpu.VMEM((CB, D), jnp.float32),
            pltpu.VMEM((CB, D), jnp.float32),
            pltpu.VMEM((negs_per_chunk, D), jnp.float32),
            pltpu.VMEM((CB,), jnp.float32),
            pltpu.VMEM((negs_per_chunk,), jnp.float32),
            pltpu.SemaphoreType.DMA,
        ],
        compiler_params=pltpu.CompilerParams(use_tc_tiling_on_sc=False),
    )
    def sc_call(in_w_hbm, out_w_hbm, cidx_hbm, ctxidx_hbm, negidx_hbm,
                pos_hbm, neg_hbm,
                cidx_v, ctxidx_v, negidx_v, crows_v, ctxrows_v, negrows_v,
                pos_v, neg_v, sem):
        wid = lax.axis_index("s") * NC + lax.axis_index("c")
        lane = lax.iota(jnp.int32, L)
        perms = [(lane ^ (1 << e))[:, None] for e in range(4)]
        gdims = lax.GatherDimensionNumbers(
            offset_dims=(), collapsed_slice_dims=(0,), start_index_map=(0,)
        )

        def lane_sum(p):
            for pm in perms:
                p = p + lax.gather(
                    p, pm, gdims, (1,),
                    mode=lax.GatherScatterMode.PROMISE_IN_BOUNDS,
                )
            return p

        def chunk_body(t, carry):
            base = wid * b_per_w + t * CB
            basek = base * K

            pltpu.sync_copy(cidx_hbm.at[pl.ds(base, CB)], cidx_v)
            pltpu.sync_copy(ctxidx_hbm.at[pl.ds(base, CB)], ctxidx_v)
            pltpu.sync_copy(negidx_hbm.at[pl.ds(basek, negs_per_chunk)], negidx_v)

            copies = [
                pltpu.async_copy(in_w_hbm.at[cidx_v], crows_v, sem),
                pltpu.async_copy(out_w_hbm.at[ctxidx_v], ctxrows_v, sem),
            ]
            for j in range(n_sub):
                copies.append(
                    pltpu.async_copy(
                        out_w_hbm.at[negidx_v.at[pl.ds(j * NEG_SUB, NEG_SUB)]],
                        negrows_v.at[pl.ds(j * NEG_SUB, NEG_SUB)],
                        sem,
                    )
                )
            for c in copies:
                c.wait()

            def dot4(rref, q, c0, c1, c2, c3):
                return (rref[q, pl.ds(0, L)] * c0
                        + rref[q, pl.ds(L, L)] * c1
                        + rref[q, pl.ds(2 * L, L)] * c2
                        + rref[q, pl.ds(3 * L, L)] * c3)

            def grp_body(g, carry2):
                pos_acc = jnp.zeros((L,), jnp.float32)

                def b_body(u, pacc):
                    bi = g * L + u
                    c0 = crows_v[bi, pl.ds(0, L)]
                    c1 = crows_v[bi, pl.ds(L, L)]
                    c2 = crows_v[bi, pl.ds(2 * L, L)]
                    c3 = crows_v[bi, pl.ds(3 * L, L)]
                    sp = lane_sum(dot4(ctxrows_v, bi, c0, c1, c2, c3))
                    pacc = jnp.where(lane == u, sp, pacc)
                    q0 = bi * K
                    acc0 = jnp.zeros((L,), jnp.float32)
                    acc1 = jnp.zeros((L,), jnp.float32)
                    for r in range(K):
                        s = lane_sum(dot4(negrows_v, q0 + r, c0, c1, c2, c3))
                        if r < L:
                            acc0 = jnp.where(lane == r, s, acc0)
                        if r >= K - L:
                            acc1 = jnp.where(lane == (r - (K - L)), s, acc1)
                    neg_v[pl.ds(q0, L)] = acc0
                    neg_v[pl.ds(q0 + (K - L), L)] = acc1
                    return pacc

                pos_acc = lax.fori_loop(0, L, b_body, pos_acc, unroll=False)
                pos_v[pl.ds(g * L, L)] = pos_acc
                return carry2

            lax.fori_loop(0, n_grp, grp_body, 0, unroll=False)

            pltpu.sync_copy(pos_v, pos_hbm.at[pl.ds(base, CB)])
            pltpu.sync_copy(neg_v, neg_hbm.at[pl.ds(basek, negs_per_chunk)])
            return carry

        lax.fori_loop(0, n_chunks, chunk_body, 0, unroll=False)

    return sc_call


def kernel(center_words, context_words, neg_samples, in_embed_w, out_embed_w):
    B = center_words.shape[0]
    V, D = in_embed_w.shape
    sc_call = _make_sc_call(B, V, D)
    pos, neg = sc_call(
        in_embed_w,
        out_embed_w,
        center_words.astype(jnp.int32),
        context_words.astype(jnp.int32),
        neg_samples.astype(jnp.int32).reshape(-1),
    )
    return pos.reshape(B, 1), neg.reshape(B, K)
